# baseline (speedup 1.0000x reference)
import jax
import jax.numpy as jnp
from jax.experimental import pallas as pl


def _pos_kernel(col_ref, row_ref, out_ref):
    j = pl.program_id(1)
    nx = pl.num_programs(1) // 2
    cblk = out_ref.shape[1]
    h = out_ref.shape[2]
    w = out_ref.shape[3]

    @pl.when(j < nx)
    def _x_part():
        xt = col_ref[...].T
        out_ref[...] = jnp.broadcast_to(xt[None, :, None, :], (1, cblk, h, w))

    @pl.when(j >= nx)
    def _y_part():
        yt = row_ref[...].T
        out_ref[...] = jnp.broadcast_to(yt[None, :, :, None], (1, cblk, h, w))


def kernel(pixel_values, row_weight, col_weight):
    batch = pixel_values.shape[0]
    height, width = pixel_values.shape[-2], pixel_values.shape[-1]
    embed_dim = row_weight.shape[1]

    cblk = 128
    nx = embed_dim // cblk

    out = pl.pallas_call(
        _pos_kernel,
        grid=(batch, 2 * nx),
        in_specs=[
            pl.BlockSpec((width, cblk), lambda b, j: (0, j % nx)),
            pl.BlockSpec((height, cblk), lambda b, j: (0, j % nx)),
        ],
        out_specs=pl.BlockSpec(
            (1, cblk, height, width), lambda b, j: (b, j, 0, 0)
        ),
        out_shape=jax.ShapeDtypeStruct(
            (batch, 2 * embed_dim, height, width), jnp.float32
        ),
    )(col_weight, row_weight)
    return out
